# baseline (device time: 33396 ns/iter reference)
import jax
import jax.numpy as jnp
from jax import lax
from jax.experimental import pallas as pl
from jax.experimental.pallas import tpu as pltpu

N_DEV = 16
NP = 4
NZ = 4


def kernel(x):
    m, n = x.shape
    chunk = m // N_DEV

    def body(x_ref, out_ref, stage_ref, pr_ref, crs_ref, cr_ref, cag_ref,
             pag_ref, pr_ssem, pr_rsem, cr_ssem, cr_rsem, ca_ssem, ca_rsem,
             pa_ssem, pa_rsem):
        my = lax.axis_index("i")
        z = my // NP
        p = my % NP

        barrier_sem = pltpu.get_barrier_semaphore()
        for o in range(1, N_DEV):
            pl.semaphore_signal(
                barrier_sem, inc=1,
                device_id=((my + o) % N_DEV,),
                device_id_type=pl.DeviceIdType.MESH,
            )
        pl.semaphore_wait(barrier_sem, N_DEV - 1)

        stage_ref[...] = x_ref[...].reshape(N_DEV, chunk, n).astype(jnp.bfloat16)

        sends = []

        for o in range(1, NP):
            p2 = (p + o) % NP
            rdma = pltpu.make_async_remote_copy(
                src_ref=stage_ref.at[pl.ds(NP * p2, NP)],
                dst_ref=pr_ref.at[p],
                send_sem=pr_ssem.at[p2],
                recv_sem=pr_rsem.at[p],
                device_id=(NP * z + p2,),
                device_id_type=pl.DeviceIdType.MESH,
            )
            rdma.start()
            sends.append(rdma)
        pr_ref[pl.ds(p, 1)] = stage_ref[pl.ds(NP * p, NP)].reshape(1, NP, chunk, n)
        for o in range(1, NP):
            ps = (p + o) % NP
            pltpu.make_async_remote_copy(
                src_ref=stage_ref.at[pl.ds(0, NP)],
                dst_ref=pr_ref.at[ps],
                send_sem=pr_ssem.at[0],
                recv_sem=pr_rsem.at[ps],
                device_id=(my,),
                device_id_type=pl.DeviceIdType.MESH,
            ).wait_recv()
        plane_part = jnp.sum(pr_ref[...].astype(jnp.float32), axis=0)
        crs_ref[...] = plane_part.astype(jnp.bfloat16)

        for o in range(1, NZ):
            z2 = (z + o) % NZ
            rdma = pltpu.make_async_remote_copy(
                src_ref=crs_ref.at[z2],
                dst_ref=cr_ref.at[z],
                send_sem=cr_ssem.at[z2],
                recv_sem=cr_rsem.at[z],
                device_id=(NP * z2 + p,),
                device_id_type=pl.DeviceIdType.MESH,
            )
            rdma.start()
            sends.append(rdma)
        cr_ref[pl.ds(z, 1)] = crs_ref[pl.ds(z, 1)]
        for o in range(1, NZ):
            zs = (z + o) % NZ
            pltpu.make_async_remote_copy(
                src_ref=crs_ref.at[0],
                dst_ref=cr_ref.at[zs],
                send_sem=cr_ssem.at[0],
                recv_sem=cr_rsem.at[zs],
                device_id=(my,),
                device_id_type=pl.DeviceIdType.MESH,
            ).wait_recv()
        reduced = jnp.sum(cr_ref[...].astype(jnp.float32), axis=0)
        cag_ref[pl.ds(z, 1)] = reduced[None].astype(jnp.bfloat16)

        for o in range(1, NZ):
            z2 = (z + o) % NZ
            rdma = pltpu.make_async_remote_copy(
                src_ref=cag_ref.at[z],
                dst_ref=cag_ref.at[z],
                send_sem=ca_ssem.at[z2],
                recv_sem=ca_rsem.at[z],
                device_id=(NP * z2 + p,),
                device_id_type=pl.DeviceIdType.MESH,
            )
            rdma.start()
            sends.append(rdma)
        for o in range(1, NZ):
            zs = (z + o) % NZ
            pltpu.make_async_remote_copy(
                src_ref=cag_ref.at[0],
                dst_ref=cag_ref.at[zs],
                send_sem=ca_ssem.at[0],
                recv_sem=ca_rsem.at[zs],
                device_id=(my,),
                device_id_type=pl.DeviceIdType.MESH,
            ).wait_recv()

        pag_ref[pl.ds(p, 1)] = cag_ref[...].reshape(1, NZ, chunk, n)
        for o in range(1, NP):
            p2 = (p + o) % NP
            rdma = pltpu.make_async_remote_copy(
                src_ref=cag_ref,
                dst_ref=pag_ref.at[p],
                send_sem=pa_ssem.at[p2],
                recv_sem=pa_rsem.at[p],
                device_id=(NP * z + p2,),
                device_id_type=pl.DeviceIdType.MESH,
            )
            rdma.start()
            sends.append(rdma)
        for o in range(1, NP):
            ps = (p + o) % NP
            pltpu.make_async_remote_copy(
                src_ref=cag_ref,
                dst_ref=pag_ref.at[ps],
                send_sem=pa_ssem.at[0],
                recv_sem=pa_rsem.at[ps],
                device_id=(my,),
                device_id_type=pl.DeviceIdType.MESH,
            ).wait_recv()

        out_ref[...] = pag_ref[...].astype(jnp.float32).reshape(m, n)

        for rdma in sends:
            rdma.wait_send()

    return pl.pallas_call(
        body,
        out_shape=jax.ShapeDtypeStruct((m, n), jnp.float32),
        in_specs=[pl.BlockSpec(memory_space=pltpu.VMEM)],
        out_specs=pl.BlockSpec(memory_space=pltpu.VMEM),
        scratch_shapes=[
            pltpu.VMEM((N_DEV, chunk, n), jnp.bfloat16),
            pltpu.VMEM((NP, NP, chunk, n), jnp.bfloat16),
            pltpu.VMEM((NZ, chunk, n), jnp.bfloat16),
            pltpu.VMEM((NZ, chunk, n), jnp.bfloat16),
            pltpu.VMEM((NZ, chunk, n), jnp.bfloat16),
            pltpu.VMEM((NP, NZ, chunk, n), jnp.bfloat16),
            pltpu.SemaphoreType.DMA((NP,)),
            pltpu.SemaphoreType.DMA((NP,)),
            pltpu.SemaphoreType.DMA((NZ,)),
            pltpu.SemaphoreType.DMA((NZ,)),
            pltpu.SemaphoreType.DMA((NZ,)),
            pltpu.SemaphoreType.DMA((NZ,)),
            pltpu.SemaphoreType.DMA((NP,)),
            pltpu.SemaphoreType.DMA((NP,)),
        ],
        compiler_params=pltpu.CompilerParams(collective_id=0),
    )(x)


# device time: 30780 ns/iter; 1.0850x vs baseline; 1.0850x over previous
import jax
import jax.numpy as jnp
from jax import lax
from jax.experimental import pallas as pl
from jax.experimental.pallas import tpu as pltpu

N_DEV = 16


def kernel(x):
    m, n = x.shape
    chunk = m // N_DEV

    def body(x_ref, out_ref, stage_ref, rs_ref, ag_ref,
             rs_send_sems, rs_recv_sems, ag_send_sems, ag_recv_sems):
        my = lax.axis_index("i")

        stage_ref[...] = x_ref[...].reshape(N_DEV, chunk, n).astype(jnp.bfloat16)

        barrier_sem = pltpu.get_barrier_semaphore()
        for o in range(1, N_DEV):
            pl.semaphore_signal(
                barrier_sem, inc=1,
                device_id=((my + o) % N_DEV,),
                device_id_type=pl.DeviceIdType.MESH,
            )
        pl.semaphore_wait(barrier_sem, N_DEV - 1)

        rs_sends = []
        for o in range(1, N_DEV):
            dest = (my + o) % N_DEV
            rdma = pltpu.make_async_remote_copy(
                src_ref=stage_ref.at[dest],
                dst_ref=rs_ref.at[my],
                send_sem=rs_send_sems.at[o],
                recv_sem=rs_recv_sems.at[my],
                device_id=(dest,),
                device_id_type=pl.DeviceIdType.MESH,
            )
            rdma.start()
            rs_sends.append(rdma)

        acc = stage_ref[my].astype(jnp.float32)
        for o in range(1, N_DEV):
            src = (my + o) % N_DEV
            pltpu.make_async_remote_copy(
                src_ref=stage_ref.at[0],
                dst_ref=rs_ref.at[src],
                send_sem=rs_send_sems.at[0],
                recv_sem=rs_recv_sems.at[src],
                device_id=(my,),
                device_id_type=pl.DeviceIdType.MESH,
            ).wait_recv()
            acc = acc + rs_ref[src].astype(jnp.float32)

        ag_ref[pl.ds(my, 1)] = acc[None].astype(jnp.bfloat16)

        ag_sends = []
        for o in range(1, N_DEV):
            dest = (my + o) % N_DEV
            rdma = pltpu.make_async_remote_copy(
                src_ref=ag_ref.at[my],
                dst_ref=ag_ref.at[my],
                send_sem=ag_send_sems.at[o],
                recv_sem=ag_recv_sems.at[my],
                device_id=(dest,),
                device_id_type=pl.DeviceIdType.MESH,
            )
            rdma.start()
            ag_sends.append(rdma)

        out_ref[pl.ds(my * chunk, chunk)] = acc

        for o in range(1, N_DEV):
            src = (my + o) % N_DEV
            pltpu.make_async_remote_copy(
                src_ref=ag_ref.at[0],
                dst_ref=ag_ref.at[src],
                send_sem=ag_send_sems.at[0],
                recv_sem=ag_recv_sems.at[src],
                device_id=(my,),
                device_id_type=pl.DeviceIdType.MESH,
            ).wait_recv()
            out_ref[pl.ds(src * chunk, chunk)] = ag_ref[src].astype(jnp.float32)

        for rdma in rs_sends + ag_sends:
            rdma.wait_send()

    return pl.pallas_call(
        body,
        out_shape=jax.ShapeDtypeStruct((m, n), jnp.float32),
        in_specs=[pl.BlockSpec(memory_space=pltpu.VMEM)],
        out_specs=pl.BlockSpec(memory_space=pltpu.VMEM),
        scratch_shapes=[
            pltpu.VMEM((N_DEV, chunk, n), jnp.bfloat16),
            pltpu.VMEM((N_DEV, chunk, n), jnp.bfloat16),
            pltpu.VMEM((N_DEV, chunk, n), jnp.bfloat16),
            pltpu.SemaphoreType.DMA((N_DEV,)),
            pltpu.SemaphoreType.DMA((N_DEV,)),
            pltpu.SemaphoreType.DMA((N_DEV,)),
            pltpu.SemaphoreType.DMA((N_DEV,)),
        ],
        compiler_params=pltpu.CompilerParams(collective_id=0),
    )(x)
